# Initial kernel scaffold; baseline (speedup 1.0000x reference)
import jax
import jax.numpy as jnp
from jax.experimental import pallas as pl


def kernel(x, table):
    raise NotImplementedError("write your pallas kernel here")


# speedup vs baseline: 2.7415x; 2.7415x over previous
import functools

import jax
import jax.numpy as jnp
import numpy as np
from jax import lax
from jax.experimental import pallas as pl
from jax.experimental.pallas import tpu as pltpu
from jax.experimental.pallas import tpu_sc as plsc

CH = 128
L = 16


def _positional_encoding(seq_len, d_model):
    depth = d_model // 2
    angle = jnp.power(
        10000.0, jnp.arange(depth, dtype=jnp.float32) * 2.0 / jnp.float32(d_model)
    )
    pos = jnp.arange(seq_len, dtype=jnp.float32)[:, None] / angle[None, :]
    pe = jnp.concatenate(
        [jnp.sin(pos)[:, None, :], jnp.cos(pos)[:, None, :]], axis=1
    )
    return pe.reshape(seq_len, d_model)


@functools.partial(jax.jit, static_argnames=())
def kernel(x, table):
    B, S = x.shape
    V, D = table.shape
    N = B * S
    scale = float(np.sqrt(D))

    info = plsc.get_sparse_core_info()
    NC, NS = info.num_cores, info.num_subcores
    NW = NC * NS
    per_w = N // NW
    nch = per_w // CH
    assert per_w * NW == N and nch * CH == per_w
    assert per_w % S == 0
    assert D % L == 0

    pe = _positional_encoding(S, D)
    reps = (S + CH + S - 1) // S
    pe_ext = jnp.tile(pe, (reps, 1))[: S + CH]

    xf2 = x.reshape(N // CH, CH)

    mesh = plsc.VectorSubcoreMesh(core_axis_name="c", subcore_axis_name="s")

    @functools.partial(
        pl.kernel,
        mesh=mesh,
        out_type=jax.ShapeDtypeStruct((N, D), jnp.float32),
        compiler_params=pltpu.CompilerParams(use_tc_tiling_on_sc=False),
        scratch_types=[
            pltpu.VMEM((nch, CH), jnp.int32),
            pltpu.VMEM((S + CH, D), jnp.float32),
            pltpu.VMEM((CH, D), jnp.float32),
            pltpu.VMEM((CH, D), jnp.float32),
            pltpu.VMEM((CH, D), jnp.float32),
            pltpu.VMEM((CH, D), jnp.float32),
            pltpu.SemaphoreType.DMA,
            pltpu.SemaphoreType.DMA,
            pltpu.SemaphoreType.DMA,
            pltpu.SemaphoreType.DMA,
        ],
    )
    def sc_kernel(
        xf2_hbm, table_hbm, pe_hbm, out_hbm,
        idx_all, pe_v, rows0, rows1, outb0, outb1, g0, g1, s0, s1,
    ):
        wid = lax.axis_index("s") * NC + lax.axis_index("c")
        base_row = wid * per_w
        base_chunk = wid * nch

        pltpu.sync_copy(xf2_hbm.at[pl.ds(base_chunk, nch)], idx_all)
        pltpu.sync_copy(pe_hbm, pe_v)

        def gather_start(j, rowsb, gsem):
            pltpu.make_async_copy(
                table_hbm.at[idx_all.at[j]], rowsb, gsem
            ).start()

        def gather_wait(rowsb, gsem):
            pltpu.make_async_copy(
                table_hbm.at[idx_all.at[0]], rowsb, gsem
            ).wait()

        def scatter_start(j, outbb, ssem):
            row0 = base_row + j * CH
            pltpu.make_async_copy(
                outbb, out_hbm.at[pl.ds(row0, CH)], ssem
            ).start()

        def scatter_wait(outbb, ssem):
            pltpu.make_async_copy(
                outbb, out_hbm.at[pl.ds(base_row, CH)], ssem
            ).wait()

        def process(j, rowsb, outbb, gsem, ssem, first):
            gather_wait(rowsb, gsem)
            if not first:
                scatter_wait(outbb, ssem)
            off = lax.rem(j * CH, S)

            def cbody(r, carry):
                for c in range(D // L):
                    sl = pl.ds(c * L, L)
                    outbb[r, sl] = rowsb[r, sl] * scale + pe_v[off + r, sl]
                return carry

            lax.fori_loop(0, CH, cbody, 0)
            gather_start(lax.rem(j + 2, nch), rowsb, gsem)
            scatter_start(j, outbb, ssem)

        gather_start(0, rows0, g0)
        gather_start(1, rows1, g1)
        process(jnp.int32(0), rows0, outb0, g0, s0, True)
        process(jnp.int32(1), rows1, outb1, g1, s1, True)

        def tbody(t, carry):
            process(2 * t, rows0, outb0, g0, s0, False)
            process(2 * t + 1, rows1, outb1, g1, s1, False)
            return carry

        lax.fori_loop(1, nch // 2, tbody, 0)

        gather_wait(rows0, g0)
        gather_wait(rows1, g1)
        scatter_wait(outb0, s0)
        scatter_wait(outb1, s1)

    out = sc_kernel(xf2, table, pe_ext)
    return out.reshape(B, S, D)


# speedup vs baseline: 2.7649x; 1.0085x over previous
import functools

import jax
import jax.numpy as jnp
import numpy as np
from jax import lax
from jax.experimental import pallas as pl
from jax.experimental.pallas import tpu as pltpu
from jax.experimental.pallas import tpu_sc as plsc

CH = 128
L = 16


def _positional_encoding(seq_len, d_model):
    depth = d_model // 2
    angle = jnp.power(
        10000.0, jnp.arange(depth, dtype=jnp.float32) * 2.0 / jnp.float32(d_model)
    )
    pos = jnp.arange(seq_len, dtype=jnp.float32)[:, None] / angle[None, :]
    pe = jnp.concatenate(
        [jnp.sin(pos)[:, None, :], jnp.cos(pos)[:, None, :]], axis=1
    )
    return pe.reshape(seq_len, d_model)


@functools.partial(jax.jit, static_argnames=())
def kernel(x, table):
    B, S = x.shape
    V, D = table.shape
    N = B * S
    scale = float(np.sqrt(D))

    info = plsc.get_sparse_core_info()
    NC, NS = info.num_cores, info.num_subcores
    NW = NC * NS
    per_w = N // NW
    nch = per_w // CH
    assert per_w * NW == N and nch * CH == per_w
    assert per_w % S == 0
    assert D % L == 0

    pe = _positional_encoding(S, D)
    reps = (S + CH + S - 1) // S
    pe_ext = jnp.tile(pe, (reps, 1))[: S + CH]

    xf2 = x.reshape(N // CH, CH)

    mesh = plsc.VectorSubcoreMesh(core_axis_name="c", subcore_axis_name="s")

    @functools.partial(
        pl.kernel,
        mesh=mesh,
        out_type=jax.ShapeDtypeStruct((N, D), jnp.float32),
        compiler_params=pltpu.CompilerParams(use_tc_tiling_on_sc=False),
        scratch_types=[
            pltpu.VMEM((nch, CH), jnp.int32),
            pltpu.VMEM((S + CH, D), jnp.float32),
            pltpu.VMEM((CH, D), jnp.float32),
            pltpu.VMEM((CH, D), jnp.float32),
            pltpu.VMEM((CH, D), jnp.float32),
            pltpu.VMEM((CH, D), jnp.float32),
            pltpu.SemaphoreType.DMA,
            pltpu.SemaphoreType.DMA,
            pltpu.SemaphoreType.DMA,
            pltpu.SemaphoreType.DMA,
        ],
    )
    def sc_kernel(
        xf2_hbm, table_hbm, pe_hbm, out_hbm,
        idx_all, pe_v, rows0, rows1, outb0, outb1, g0, g1, s0, s1,
    ):
        wid = lax.axis_index("s") * NC + lax.axis_index("c")
        base_row = wid * per_w
        base_chunk = wid * nch

        pltpu.sync_copy(xf2_hbm.at[pl.ds(base_chunk, nch)], idx_all)
        pltpu.sync_copy(pe_hbm, pe_v)

        def gather_start(j, rowsb, gsem):
            pltpu.make_async_copy(
                table_hbm.at[idx_all.at[j]], rowsb, gsem
            ).start()

        def gather_wait(rowsb, gsem):
            pltpu.make_async_copy(
                table_hbm.at[idx_all.at[0]], rowsb, gsem
            ).wait()

        def scatter_start(j, outbb, ssem):
            row0 = base_row + j * CH
            pltpu.make_async_copy(
                outbb, out_hbm.at[pl.ds(row0, CH)], ssem
            ).start()

        def scatter_wait(outbb, ssem):
            pltpu.make_async_copy(
                outbb, out_hbm.at[pl.ds(base_row, CH)], ssem
            ).wait()

        def process(j, rowsb, outbb, gsem, ssem, first):
            gather_wait(rowsb, gsem)
            if not first:
                scatter_wait(outbb, ssem)
            off = lax.rem(j * CH, S)

            UNROLL = 4

            def cbody(rr, carry):
                r0 = rr * UNROLL
                for u in range(UNROLL):
                    r = r0 + u
                    for c in range(D // L):
                        sl = pl.ds(c * L, L)
                        outbb[r, sl] = rowsb[r, sl] * scale + pe_v[off + r, sl]
                return carry

            lax.fori_loop(0, CH // UNROLL, cbody, 0)
            gather_start(lax.rem(j + 2, nch), rowsb, gsem)
            scatter_start(j, outbb, ssem)

        gather_start(0, rows0, g0)
        gather_start(1, rows1, g1)
        process(jnp.int32(0), rows0, outb0, g0, s0, True)
        process(jnp.int32(1), rows1, outb1, g1, s1, True)

        def tbody(t, carry):
            process(2 * t, rows0, outb0, g0, s0, False)
            process(2 * t + 1, rows1, outb1, g1, s1, False)
            return carry

        lax.fori_loop(1, nch // 2, tbody, 0)

        gather_wait(rows0, g0)
        gather_wait(rows1, g1)
        scatter_wait(outb0, s0)
        scatter_wait(outb1, s1)

    out = sc_kernel(xf2, table, pe_ext)
    return out.reshape(B, S, D)


# speedup vs baseline: 2.7795x; 1.0053x over previous
import functools

import jax
import jax.numpy as jnp
import numpy as np
from jax import lax
from jax.experimental import pallas as pl
from jax.experimental.pallas import tpu as pltpu
from jax.experimental.pallas import tpu_sc as plsc

CH = 128
L = 16
RING = 5


def _positional_encoding(seq_len, d_model):
    depth = d_model // 2
    angle = jnp.power(
        10000.0, jnp.arange(depth, dtype=jnp.float32) * 2.0 / jnp.float32(d_model)
    )
    pos = jnp.arange(seq_len, dtype=jnp.float32)[:, None] / angle[None, :]
    pe = jnp.concatenate(
        [jnp.sin(pos)[:, None, :], jnp.cos(pos)[:, None, :]], axis=1
    )
    return pe.reshape(seq_len, d_model)


@functools.partial(jax.jit, static_argnames=())
def kernel(x, table):
    B, S = x.shape
    V, D = table.shape
    N = B * S
    scale = float(np.sqrt(D))

    info = plsc.get_sparse_core_info()
    NC, NS = info.num_cores, info.num_subcores
    NW = NC * NS
    per_w = N // NW
    nch = per_w // CH
    assert per_w * NW == N and nch * CH == per_w
    assert per_w % S == 0
    assert D % L == 0
    assert nch % RING == 0

    pe = _positional_encoding(S, D)
    reps = (S + CH + S - 1) // S
    pe_ext = jnp.tile(pe, (reps, 1))[: S + CH]

    xf2 = x.reshape(N // CH, CH)

    mesh = plsc.VectorSubcoreMesh(core_axis_name="c", subcore_axis_name="s")

    @functools.partial(
        pl.kernel,
        mesh=mesh,
        out_type=jax.ShapeDtypeStruct((N, D), jnp.float32),
        compiler_params=pltpu.CompilerParams(use_tc_tiling_on_sc=False),
        scratch_types=(
            [
                pltpu.VMEM((nch, CH), jnp.int32),
                pltpu.VMEM((S + CH, D), jnp.float32),
            ]
            + [pltpu.VMEM((CH, D), jnp.float32)] * RING
            + [pltpu.VMEM((CH, D), jnp.float32)] * RING
            + [pltpu.SemaphoreType.DMA] * (2 * RING)
        ),
    )
    def sc_kernel(xf2_hbm, table_hbm, pe_hbm, out_hbm, idx_all, pe_v, *bufs):
        rows = bufs[:RING]
        outb = bufs[RING : 2 * RING]
        gsems = bufs[2 * RING : 3 * RING]
        ssems = bufs[3 * RING : 4 * RING]
        wid = lax.axis_index("s") * NC + lax.axis_index("c")
        base_row = wid * per_w
        base_chunk = wid * nch

        pltpu.sync_copy(xf2_hbm.at[pl.ds(base_chunk, nch)], idx_all)
        pltpu.sync_copy(pe_hbm, pe_v)

        def gather_start(j, rowsb, gsem):
            pltpu.make_async_copy(
                table_hbm.at[idx_all.at[j]], rowsb, gsem
            ).start()

        def gather_wait(rowsb, gsem):
            pltpu.make_async_copy(
                table_hbm.at[idx_all.at[0]], rowsb, gsem
            ).wait()

        def scatter_start(j, outbb, ssem):
            row0 = base_row + j * CH
            pltpu.make_async_copy(
                outbb, out_hbm.at[pl.ds(row0, CH)], ssem
            ).start()

        def scatter_wait(outbb, ssem):
            pltpu.make_async_copy(
                outbb, out_hbm.at[pl.ds(base_row, CH)], ssem
            ).wait()

        def process(j, rowsb, outbb, gsem, ssem, first):
            gather_wait(rowsb, gsem)
            if not first:
                scatter_wait(outbb, ssem)
            off = lax.rem(j * CH, S)

            UNROLL = 4

            def cbody(rr, carry):
                r0 = rr * UNROLL
                for u in range(UNROLL):
                    r = r0 + u
                    for c in range(D // L):
                        sl = pl.ds(c * L, L)
                        outbb[r, sl] = rowsb[r, sl] * scale + pe_v[off + r, sl]
                return carry

            lax.fori_loop(0, CH // UNROLL, cbody, 0)
            gather_start(lax.rem(j + RING, nch), rowsb, gsem)
            scatter_start(j, outbb, ssem)

        for p in range(RING):
            gather_start(p, rows[p], gsems[p])
        for p in range(RING):
            process(jnp.int32(p), rows[p], outb[p], gsems[p], ssems[p], True)

        def tbody(t, carry):
            for p in range(RING):
                process(RING * t + p, rows[p], outb[p], gsems[p], ssems[p], False)
            return carry

        lax.fori_loop(1, nch // RING, tbody, 0)

        for p in range(RING):
            gather_wait(rows[p], gsems[p])
        for p in range(RING):
            scatter_wait(outb[p], ssems[p])

    out = sc_kernel(xf2, table, pe_ext)
    return out.reshape(B, S, D)


# speedup vs baseline: 4.2171x; 1.5172x over previous
import functools

import jax
import jax.numpy as jnp
import numpy as np
from jax import lax
from jax.experimental import pallas as pl
from jax.experimental.pallas import tpu as pltpu
from jax.experimental.pallas import tpu_sc as plsc

L = 16
RING = 4
CH0 = 104


def _positional_encoding(seq_len, d_model):
    depth = d_model // 2
    angle = jnp.power(
        10000.0, jnp.arange(depth, dtype=jnp.float32) * 2.0 / jnp.float32(d_model)
    )
    pos = jnp.arange(seq_len, dtype=jnp.float32)[:, None] / angle[None, :]
    pe = jnp.concatenate(
        [jnp.sin(pos)[:, None, :], jnp.cos(pos)[:, None, :]], axis=1
    )
    return pe.reshape(seq_len, d_model)


@jax.jit
def kernel(x, table):
    B, S = x.shape
    V, D = table.shape
    scale = float(np.sqrt(D))

    info = plsc.get_sparse_core_info()
    NC, NS = info.num_cores, info.num_subcores
    NW = NC * NS
    seq_per_w = B // NW
    ch1 = S - CH0
    assert seq_per_w * NW == B
    assert seq_per_w % RING == 0
    assert D % L == 0
    assert S % 8 == 0 and CH0 % 8 == 0 and 0 < ch1 <= 128

    pe = _positional_encoding(S, D)

    mesh = plsc.VectorSubcoreMesh(core_axis_name="c", subcore_axis_name="s")

    @functools.partial(
        pl.kernel,
        mesh=mesh,
        out_type=jax.ShapeDtypeStruct((B, S, D), jnp.float32),
        compiler_params=pltpu.CompilerParams(use_tc_tiling_on_sc=False),
        scratch_types=(
            [pltpu.VMEM((S, D), jnp.float32)]
            + [pltpu.VMEM((S,), jnp.int32)] * RING
            + [pltpu.VMEM((S, D), jnp.float32)] * RING
            + [pltpu.VMEM((S, D), jnp.float32)] * RING
            + [pltpu.SemaphoreType.DMA] * (3 * RING)
        ),
    )
    def sc_kernel(x_hbm, table_hbm, pe_hbm, out_hbm, pe_v, *bufs):
        idxb = bufs[:RING]
        rows = bufs[RING : 2 * RING]
        outb = bufs[2 * RING : 3 * RING]
        isems = bufs[3 * RING : 4 * RING]
        gsems = bufs[4 * RING : 5 * RING]
        ssems = bufs[5 * RING : 6 * RING]

        wid = lax.axis_index("s") * NC + lax.axis_index("c")
        base_seq = wid * seq_per_w

        pltpu.sync_copy(pe_hbm, pe_v)

        def idx_start(s, p):
            pltpu.make_async_copy(
                x_hbm.at[base_seq + s], idxb[p], isems[p]
            ).start()

        def idx_wait(p):
            pltpu.make_async_copy(x_hbm.at[base_seq], idxb[p], isems[p]).wait()

        def gathers_start(p):
            pltpu.make_async_copy(
                table_hbm.at[idxb[p].at[pl.ds(0, CH0)]],
                rows[p].at[pl.ds(0, CH0)],
                gsems[p],
            ).start()
            pltpu.make_async_copy(
                table_hbm.at[idxb[p].at[pl.ds(CH0, ch1)]],
                rows[p].at[pl.ds(CH0, ch1)],
                gsems[p],
            ).start()

        def gathers_wait(p):
            pltpu.make_async_copy(
                table_hbm.at[idxb[p].at[pl.ds(0, CH0)]],
                rows[p].at[pl.ds(0, CH0)],
                gsems[p],
            ).wait()
            pltpu.make_async_copy(
                table_hbm.at[idxb[p].at[pl.ds(CH0, ch1)]],
                rows[p].at[pl.ds(CH0, ch1)],
                gsems[p],
            ).wait()

        def scatter_start(s, p):
            pltpu.make_async_copy(
                outb[p], out_hbm.at[base_seq + s], ssems[p]
            ).start()

        def scatter_wait(p):
            pltpu.make_async_copy(
                outb[p], out_hbm.at[base_seq], ssems[p]
            ).wait()

        UNROLL = 4

        def process(s, p, first):
            gathers_wait(p)
            sn = lax.rem(s + RING, seq_per_w)
            idx_start(sn, p)
            if not first:
                scatter_wait(p)

            def cbody(rr, carry):
                r0 = rr * UNROLL
                for u in range(UNROLL):
                    r = r0 + u
                    for c in range(D // L):
                        sl = pl.ds(c * L, L)
                        outb[p][r, sl] = rows[p][r, sl] * scale + pe_v[r, sl]
                return carry

            lax.fori_loop(0, S // UNROLL, cbody, 0)
            idx_wait(p)
            gathers_start(p)
            scatter_start(s, p)

        for p in range(RING):
            idx_start(jnp.int32(p), p)
        for p in range(RING):
            idx_wait(p)
            gathers_start(p)
        for p in range(RING):
            process(jnp.int32(p), p, True)

        def tbody(t, carry):
            for p in range(RING):
                process(RING * t + p, p, False)
            return carry

        lax.fori_loop(1, seq_per_w // RING, tbody, 0)

        for p in range(RING):
            gathers_wait(p)
        for p in range(RING):
            scatter_wait(p)

    return sc_kernel(x, table, pe)


# speedup vs baseline: 4.3654x; 1.0352x over previous
import functools

import jax
import jax.numpy as jnp
import numpy as np
from jax import lax
from jax.experimental import layout as jlayout
from jax.experimental import pallas as pl
from jax.experimental.pallas import tpu as pltpu
from jax.experimental.pallas import tpu_sc as plsc

L = 16
RING = 4
CH0 = 104


def _positional_encoding(seq_len, d_model):
    depth = d_model // 2
    angle = jnp.power(
        10000.0, jnp.arange(depth, dtype=jnp.float32) * 2.0 / jnp.float32(d_model)
    )
    pos = jnp.arange(seq_len, dtype=jnp.float32)[:, None] / angle[None, :]
    pe = jnp.concatenate(
        [jnp.sin(pos)[:, None, :], jnp.cos(pos)[:, None, :]], axis=1
    )
    return pe.reshape(seq_len, d_model)


@jax.jit
def kernel(x, table):
    B, S = x.shape
    V, D = table.shape
    scale = float(np.sqrt(D))

    info = plsc.get_sparse_core_info()
    NC, NS = info.num_cores, info.num_subcores
    NW = NC * NS
    seq_per_w = B // NW
    ch1 = S - CH0
    assert seq_per_w * NW == B
    assert seq_per_w % RING == 0
    assert D % L == 0
    assert S % 8 == 0 and CH0 % 8 == 0 and 0 < ch1 <= 128

    pe = _positional_encoding(S, D)

    def _linear(a):
        lay = jlayout.Layout(tuple(range(a.ndim)), tiling=())
        return jlayout.with_layout_constraint(a, lay)

    x = _linear(x)
    table = _linear(table)
    pe = _linear(pe)

    mesh = plsc.VectorSubcoreMesh(core_axis_name="c", subcore_axis_name="s")

    @functools.partial(
        pl.kernel,
        mesh=mesh,
        out_type=jax.ShapeDtypeStruct((B, S, D), jnp.float32),
        compiler_params=pltpu.CompilerParams(use_tc_tiling_on_sc=False),
        scratch_types=(
            [pltpu.VMEM((S, D), jnp.float32)]
            + [pltpu.VMEM((S,), jnp.int32)] * RING
            + [pltpu.VMEM((S, D), jnp.float32)] * RING
            + [pltpu.VMEM((S, D), jnp.float32)] * RING
            + [pltpu.SemaphoreType.DMA] * (3 * RING)
        ),
    )
    def sc_kernel(x_hbm, table_hbm, pe_hbm, out_hbm, pe_v, *bufs):
        idxb = bufs[:RING]
        rows = bufs[RING : 2 * RING]
        outb = bufs[2 * RING : 3 * RING]
        isems = bufs[3 * RING : 4 * RING]
        gsems = bufs[4 * RING : 5 * RING]
        ssems = bufs[5 * RING : 6 * RING]

        wid = lax.axis_index("s") * NC + lax.axis_index("c")
        base_seq = wid * seq_per_w

        pltpu.sync_copy(pe_hbm, pe_v)

        def idx_start(s, p):
            pltpu.make_async_copy(
                x_hbm.at[base_seq + s], idxb[p], isems[p]
            ).start()

        def idx_wait(p):
            pltpu.make_async_copy(x_hbm.at[base_seq], idxb[p], isems[p]).wait()

        def gathers_start(p):
            pltpu.make_async_copy(
                table_hbm.at[idxb[p].at[pl.ds(0, CH0)]],
                rows[p].at[pl.ds(0, CH0)],
                gsems[p],
            ).start()
            pltpu.make_async_copy(
                table_hbm.at[idxb[p].at[pl.ds(CH0, ch1)]],
                rows[p].at[pl.ds(CH0, ch1)],
                gsems[p],
            ).start()

        def gathers_wait(p):
            pltpu.make_async_copy(
                table_hbm.at[idxb[p].at[pl.ds(0, CH0)]],
                rows[p].at[pl.ds(0, CH0)],
                gsems[p],
            ).wait()
            pltpu.make_async_copy(
                table_hbm.at[idxb[p].at[pl.ds(CH0, ch1)]],
                rows[p].at[pl.ds(CH0, ch1)],
                gsems[p],
            ).wait()

        def scatter_start(s, p):
            pltpu.make_async_copy(
                outb[p], out_hbm.at[base_seq + s], ssems[p]
            ).start()

        def scatter_wait(p):
            pltpu.make_async_copy(
                outb[p], out_hbm.at[base_seq], ssems[p]
            ).wait()

        UNROLL = 4

        def process(s, p, first):
            gathers_wait(p)
            sn = lax.rem(s + RING, seq_per_w)
            idx_start(sn, p)
            if not first:
                scatter_wait(p)

            def cbody(rr, carry):
                r0 = rr * UNROLL
                for u in range(UNROLL):
                    r = r0 + u
                    for c in range(D // L):
                        sl = pl.ds(c * L, L)
                        outb[p][r, sl] = rows[p][r, sl] * scale + pe_v[r, sl]
                return carry

            lax.fori_loop(0, S // UNROLL, cbody, 0)
            idx_wait(p)
            gathers_start(p)
            scatter_start(s, p)

        for p in range(RING):
            idx_start(jnp.int32(p), p)
        for p in range(RING):
            idx_wait(p)
            gathers_start(p)
        for p in range(RING):
            process(jnp.int32(p), p, True)

        def tbody(t, carry):
            for p in range(RING):
                process(RING * t + p, p, False)
            return carry

        lax.fori_loop(1, seq_per_w // RING, tbody, 0)

        for p in range(RING):
            gathers_wait(p)
        for p in range(RING):
            scatter_wait(p)

    return sc_kernel(x, table, pe)
